# baseline (speedup 1.0000x reference)
import functools

import jax
import jax.numpy as jnp
from jax import lax
from jax.experimental import pallas as pl
from jax.experimental.pallas import tpu as pltpu
from jax.experimental.pallas import tpu_sc as plsc

N_NODES = 10000
N_EDGES = 320000
D = 128

NC = 2
NS = 16
NW = NC * NS
NP = 10240
EPT = N_EDGES // NW
K = 125
NCHUNK = EPT // K
NB = 20
NBUF = 2
RPT = NP // NS
ZCH = [80] * (RPT // 80)


def _make_seg_sum():
  mesh = plsc.VectorSubcoreMesh(core_axis_name="c", subcore_axis_name="s",
                                num_cores=NC, num_subcores=NS)
  scratch = [
      pltpu.VMEM((2, NB, K), jnp.int32),
      pltpu.VMEM((2, NB, K), jnp.int32),
      pltpu.VMEM((NBUF, K, D), jnp.float32),
      pltpu.VMEM_SHARED((NP, D), jnp.float32),
  ] + [pltpu.SemaphoreType.DMA] * (2 * NBUF + 2)

  def body(x_hbm, ei_hbm, zrow_hbm, part_hbm,
           src_v, dst_v, rows_v, acc, *sems):
    c = lax.axis_index("c")
    s = lax.axis_index("s")
    wid = s * NC + c
    gsems = sems[:NBUF]
    ssems = sems[NBUF:2 * NBUF]
    isems = sems[2 * NBUF:]

    pltpu.sync_copy(zrow_hbm, rows_v.at[0])
    r = 0
    for z in ZCH:
      pltpu.sync_copy(rows_v.at[0, pl.ds(0, z)],
                      acc.at[pl.ds(s * RPT + r, z)])
      r += z
    plsc.subcore_barrier()

    def iload(blk):
      p = blk % 2
      pltpu.async_copy(ei_hbm.at[0, wid, blk], src_v.at[p], isems[p])
      pltpu.async_copy(ei_hbm.at[1, wid, blk], dst_v.at[p], isems[p])

    def iwait(blk):
      p = blk % 2
      pltpu.make_async_copy(ei_hbm.at[0, wid, 0], src_v.at[p],
                            isems[p]).wait()
      pltpu.make_async_copy(ei_hbm.at[1, wid, 0], dst_v.at[p],
                            isems[p]).wait()

    def gather(i, b):
      blk, loc = divmod(i, NB)
      pltpu.async_copy(x_hbm.at[src_v.at[blk % 2, loc]], rows_v.at[b],
                       gsems[b])

    def gwait(b):
      pltpu.make_async_copy(x_hbm.at[src_v.at[0, 0]], rows_v.at[b],
                            gsems[b]).wait()

    def scatter(i, b):
      blk, loc = divmod(i, NB)
      pltpu.async_copy(rows_v.at[b], acc.at[dst_v.at[blk % 2, loc]],
                       ssems[b], add=True)

    def swait(b):
      pltpu.make_async_copy(rows_v.at[b], acc.at[dst_v.at[0, 0]],
                            ssems[b]).wait()

    iload(0)
    iwait(0)
    iload(1)
    for b in range(NBUF - 1):
      gather(b, b)
    for i in range(NCHUNK):
      b = i % NBUF
      gwait(b)
      scatter(i, b)
      g = i + NBUF - 1
      if g < NCHUNK:
        if g % NB == 0:
          iwait(g // NB)
        if i > 0:
          swait(g % NBUF)
        if i % NB == 0 and 1 < i // NB + 1 < NCHUNK // NB:
          iload(i // NB + 1)
        gather(g, g % NBUF)
    for i in range(NCHUNK - NBUF, NCHUNK):
      swait(i % NBUF)

    plsc.subcore_barrier()

    r = 0
    for z in ZCH:
      pltpu.sync_copy(acc.at[pl.ds(s * RPT + r, z)],
                      rows_v.at[0, pl.ds(0, z)])
      pltpu.sync_copy(rows_v.at[0, pl.ds(0, z)],
                      part_hbm.at[c, pl.ds(s * RPT + r, z)])
      r += z

  return pl.kernel(
      body,
      out_type=jax.ShapeDtypeStruct((NC, NP, D), jnp.float32),
      mesh=mesh,
      scratch_types=scratch,
  )


@functools.lru_cache(maxsize=None)
def _get_seg_sum():
  return _make_seg_sum()


BE = 4096
EP = 327680
BM = 2000


def _deg_body(dr_ref, o_ref):
  dr = dr_ref[...]
  sub = lax.broadcasted_iota(jnp.int32, (128, BE), 0)
  qrow = lax.shift_right_logical(dr, 7)
  a_t = jnp.where(sub == qrow, 1.0, 0.0).astype(jnp.bfloat16)
  rcol = lax.bitwise_and(dr, 127)
  b_t = jnp.where(sub == rcol, 1.0, 0.0).astype(jnp.bfloat16)
  inc = lax.dot_general(a_t, b_t, (((1,), (1,)), ((), ())),
                        preferred_element_type=jnp.float32)

  @pl.when(pl.program_id(0) == 0)
  def _():
    o_ref[...] = jnp.zeros_like(o_ref)

  o_ref[...] += inc


_deg_tc = pl.pallas_call(
    _deg_body,
    grid=(EP // BE,),
    in_specs=[pl.BlockSpec((1, BE), lambda i: (0, i))],
    out_specs=pl.BlockSpec((128, 128), lambda i: (0, 0)),
    out_shape=jax.ShapeDtypeStruct((128, 128), jnp.float32),
)


def _l1_body(x_ref, p_ref, dg_ref, wl_ref, wr_ref, b_ref, o_ref):
  ssum = p_ref[0] + p_ref[1]
  deg = dg_ref[...]
  agg = ssum * (1.0 / jnp.maximum(deg, 1.0))
  y = (jnp.dot(agg, wl_ref[...], preferred_element_type=jnp.float32,
               precision=lax.Precision.HIGHEST)
       + jnp.dot(x_ref[...], wr_ref[...], preferred_element_type=jnp.float32,
                 precision=lax.Precision.HIGHEST)
       + b_ref[...])
  o_ref[...] = jnp.maximum(y, 0.0)


def _l2_body(h_ref, p_ref, dg_ref, wl_ref, wr_ref, b_ref, wo_ref, bo_ref,
             o_ref):
  ssum = p_ref[0] + p_ref[1]
  deg = dg_ref[...]
  agg = ssum * (1.0 / jnp.maximum(deg, 1.0))
  y = (jnp.dot(agg, wl_ref[...], preferred_element_type=jnp.float32,
               precision=lax.Precision.HIGHEST)
       + jnp.dot(h_ref[...], wr_ref[...], preferred_element_type=jnp.float32,
                 precision=lax.Precision.HIGHEST)
       + b_ref[...])
  h2 = jnp.maximum(y, 0.0)
  o_ref[...] = (jnp.dot(h2, wo_ref[...], preferred_element_type=jnp.float32,
                        precision=lax.Precision.HIGHEST)
                + bo_ref[...])


_row_spec = pl.BlockSpec((BM, D), lambda i: (i, 0))
_part_spec = pl.BlockSpec((NC, BM, D), lambda i: (0, i, 0))
_deg_spec = pl.BlockSpec((BM, 1), lambda i: (i, 0))
_w_spec = pl.BlockSpec((D, D), lambda i: (0, 0))
_b_spec = pl.BlockSpec((1, D), lambda i: (0, 0))

_layer1_tc = pl.pallas_call(
    _l1_body,
    grid=(N_NODES // BM,),
    in_specs=[_row_spec, _part_spec, _deg_spec, _w_spec, _w_spec, _b_spec],
    out_specs=_row_spec,
    out_shape=jax.ShapeDtypeStruct((N_NODES, D), jnp.float32),
)

_layer2_tc = pl.pallas_call(
    _l2_body,
    grid=(N_NODES // BM,),
    in_specs=[_row_spec, _part_spec, _deg_spec, _w_spec, _w_spec, _b_spec,
              _w_spec, _b_spec],
    out_specs=_row_spec,
    out_shape=jax.ShapeDtypeStruct((N_NODES, D), jnp.float32),
)


def kernel(x, edge_index, W1l, W1r, b1, W2l, W2r, b2, Wlin, blin):
  zrow = jnp.zeros((K, D), jnp.float32)
  dst_pad = jnp.pad(edge_index[1], (0, EP - N_EDGES), constant_values=NP)
  b1r = b1.reshape(1, D)
  b2r = b2.reshape(1, D)
  blinr = blin.reshape(1, D)

  ei5 = edge_index.reshape(2, NW, NCHUNK // NB, NB, K)
  part1 = _get_seg_sum()(x, ei5, zrow)
  deg = _deg_tc(dst_pad.reshape(1, EP))
  deg_col = deg.reshape(128 * 128, 1)
  h = _layer1_tc(x, part1, deg_col, W1l, W1r, b1r)
  part2 = _get_seg_sum()(h, ei5, zrow)
  return _layer2_tc(h, part2, deg_col, W2l, W2r, b2r, Wlin, blinr)


# speedup vs baseline: 1.2109x; 1.2109x over previous
import functools

import jax
import jax.numpy as jnp
from jax import lax
from jax.experimental import pallas as pl
from jax.experimental.pallas import tpu as pltpu
from jax.experimental.pallas import tpu_sc as plsc

N_NODES = 10000
N_EDGES = 320000
D = 128

NC = 2
NS = 16
NW = NC * NS
NP = 10240
EPT = N_EDGES // NW
K = 80
NCHUNK = EPT // K
NB = 5
NBUF = 4
RPT = NP // NS
ZCH = [80] * (RPT // 80)


def _make_seg_sum():
  mesh = plsc.VectorSubcoreMesh(core_axis_name="c", subcore_axis_name="s",
                                num_cores=NC, num_subcores=NS)
  scratch = [
      pltpu.VMEM((2, NB, K), jnp.int32),
      pltpu.VMEM((2, NB, K), jnp.int32),
      pltpu.VMEM((NBUF, K, D), jnp.float32),
      pltpu.VMEM_SHARED((NP, D), jnp.float32),
  ] + [pltpu.SemaphoreType.DMA] * (2 * NBUF + 2)

  def body(x_hbm, ei_hbm, zrow_hbm, part_hbm,
           src_v, dst_v, rows_v, acc, *sems):
    c = lax.axis_index("c")
    s = lax.axis_index("s")
    wid = s * NC + c
    gsems = sems[:NBUF]
    ssems = sems[NBUF:2 * NBUF]
    isems = sems[2 * NBUF:]

    pltpu.sync_copy(zrow_hbm, rows_v.at[0])
    r = 0
    for z in ZCH:
      pltpu.sync_copy(rows_v.at[0, pl.ds(0, z)],
                      acc.at[pl.ds(s * RPT + r, z)])
      r += z
    plsc.subcore_barrier()

    def iload(blk):
      p = blk % 2
      pltpu.async_copy(ei_hbm.at[0, wid, blk], src_v.at[p], isems[p])
      pltpu.async_copy(ei_hbm.at[1, wid, blk], dst_v.at[p], isems[p])

    def iwait(blk):
      p = blk % 2
      pltpu.make_async_copy(ei_hbm.at[0, wid, 0], src_v.at[p],
                            isems[p]).wait()
      pltpu.make_async_copy(ei_hbm.at[1, wid, 0], dst_v.at[p],
                            isems[p]).wait()

    def gather(i, b):
      blk, loc = divmod(i, NB)
      pltpu.async_copy(x_hbm.at[src_v.at[blk % 2, loc]], rows_v.at[b],
                       gsems[b])

    def gwait(b):
      pltpu.make_async_copy(x_hbm.at[src_v.at[0, 0]], rows_v.at[b],
                            gsems[b]).wait()

    def scatter(i, b):
      blk, loc = divmod(i, NB)
      pltpu.async_copy(rows_v.at[b], acc.at[dst_v.at[blk % 2, loc]],
                       ssems[b], add=True)

    def swait(b):
      pltpu.make_async_copy(rows_v.at[b], acc.at[dst_v.at[0, 0]],
                            ssems[b]).wait()

    iload(0)
    iwait(0)
    iload(1)
    for b in range(NBUF - 1):
      gather(b, b)
    for i in range(NCHUNK):
      b = i % NBUF
      gwait(b)
      scatter(i, b)
      g = i + NBUF - 1
      if g < NCHUNK:
        if g % NB == 0:
          iwait(g // NB)
        if i > 0:
          swait(g % NBUF)
        if i % NB == 0 and 1 < i // NB + 1 < NCHUNK // NB:
          iload(i // NB + 1)
        gather(g, g % NBUF)
    for i in range(NCHUNK - NBUF, NCHUNK):
      swait(i % NBUF)

    plsc.subcore_barrier()

    r = 0
    for z in ZCH:
      pltpu.sync_copy(acc.at[pl.ds(s * RPT + r, z)],
                      rows_v.at[0, pl.ds(0, z)])
      pltpu.sync_copy(rows_v.at[0, pl.ds(0, z)],
                      part_hbm.at[c, pl.ds(s * RPT + r, z)])
      r += z

  return pl.kernel(
      body,
      out_type=jax.ShapeDtypeStruct((NC, NP, D), jnp.float32),
      mesh=mesh,
      scratch_types=scratch,
  )


@functools.lru_cache(maxsize=None)
def _get_seg_sum():
  return _make_seg_sum()


BE = 4096
EP = 327680
BM = 2000


def _deg_body(dr_ref, o_ref):
  dr = dr_ref[...]
  sub = lax.broadcasted_iota(jnp.int32, (128, BE), 0)
  qrow = lax.shift_right_logical(dr, 7)
  a_t = jnp.where(sub == qrow, 1.0, 0.0).astype(jnp.bfloat16)
  rcol = lax.bitwise_and(dr, 127)
  b_t = jnp.where(sub == rcol, 1.0, 0.0).astype(jnp.bfloat16)
  inc = lax.dot_general(a_t, b_t, (((1,), (1,)), ((), ())),
                        preferred_element_type=jnp.float32)

  @pl.when(pl.program_id(0) == 0)
  def _():
    o_ref[...] = jnp.zeros_like(o_ref)

  o_ref[...] += inc


_deg_tc = pl.pallas_call(
    _deg_body,
    grid=(EP // BE,),
    in_specs=[pl.BlockSpec((1, BE), lambda i: (0, i))],
    out_specs=pl.BlockSpec((128, 128), lambda i: (0, 0)),
    out_shape=jax.ShapeDtypeStruct((128, 128), jnp.float32),
)


def _l1_body(x_ref, p_ref, dg_ref, wl_ref, wr_ref, b_ref, o_ref):
  ssum = p_ref[0] + p_ref[1]
  deg = dg_ref[...]
  agg = ssum * (1.0 / jnp.maximum(deg, 1.0))
  y = (jnp.dot(agg, wl_ref[...], preferred_element_type=jnp.float32,
               precision=lax.Precision.HIGHEST)
       + jnp.dot(x_ref[...], wr_ref[...], preferred_element_type=jnp.float32,
                 precision=lax.Precision.HIGHEST)
       + b_ref[...])
  o_ref[...] = jnp.maximum(y, 0.0)


def _l2_body(h_ref, p_ref, dg_ref, wl_ref, wr_ref, b_ref, wo_ref, bo_ref,
             o_ref):
  ssum = p_ref[0] + p_ref[1]
  deg = dg_ref[...]
  agg = ssum * (1.0 / jnp.maximum(deg, 1.0))
  y = (jnp.dot(agg, wl_ref[...], preferred_element_type=jnp.float32,
               precision=lax.Precision.HIGHEST)
       + jnp.dot(h_ref[...], wr_ref[...], preferred_element_type=jnp.float32,
                 precision=lax.Precision.HIGHEST)
       + b_ref[...])
  h2 = jnp.maximum(y, 0.0)
  o_ref[...] = (jnp.dot(h2, wo_ref[...], preferred_element_type=jnp.float32,
                        precision=lax.Precision.HIGHEST)
                + bo_ref[...])


_row_spec = pl.BlockSpec((BM, D), lambda i: (i, 0))
_part_spec = pl.BlockSpec((NC, BM, D), lambda i: (0, i, 0))
_deg_spec = pl.BlockSpec((BM, 1), lambda i: (i, 0))
_w_spec = pl.BlockSpec((D, D), lambda i: (0, 0))
_b_spec = pl.BlockSpec((1, D), lambda i: (0, 0))

_layer1_tc = pl.pallas_call(
    _l1_body,
    grid=(N_NODES // BM,),
    in_specs=[_row_spec, _part_spec, _deg_spec, _w_spec, _w_spec, _b_spec],
    out_specs=_row_spec,
    out_shape=jax.ShapeDtypeStruct((N_NODES, D), jnp.float32),
)

_layer2_tc = pl.pallas_call(
    _l2_body,
    grid=(N_NODES // BM,),
    in_specs=[_row_spec, _part_spec, _deg_spec, _w_spec, _w_spec, _b_spec,
              _w_spec, _b_spec],
    out_specs=_row_spec,
    out_shape=jax.ShapeDtypeStruct((N_NODES, D), jnp.float32),
)


def kernel(x, edge_index, W1l, W1r, b1, W2l, W2r, b2, Wlin, blin):
  zrow = jnp.zeros((K, D), jnp.float32)
  dst_pad = jnp.pad(edge_index[1], (0, EP - N_EDGES), constant_values=NP)
  b1r = b1.reshape(1, D)
  b2r = b2.reshape(1, D)
  blinr = blin.reshape(1, D)

  ei5 = edge_index.reshape(2, NW, NCHUNK // NB, NB, K)
  part1 = _get_seg_sum()(x, ei5, zrow)
  deg = _deg_tc(dst_pad.reshape(1, EP))
  deg_col = deg.reshape(128 * 128, 1)
  h = _layer1_tc(x, part1, deg_col, W1l, W1r, b1r)
  part2 = _get_seg_sum()(h, ei5, zrow)
  return _layer2_tc(h, part2, deg_col, W2l, W2r, b2r, Wlin, blinr)


# speedup vs baseline: 1.2323x; 1.0176x over previous
import functools

import jax
import jax.numpy as jnp
from jax import lax
from jax.experimental import pallas as pl
from jax.experimental.pallas import tpu as pltpu
from jax.experimental.pallas import tpu_sc as plsc

N_NODES = 10000
N_EDGES = 320000
D = 128

NC = 2
NS = 16
NW = NC * NS
NP = 10240
EPT = N_EDGES // NW
K = 80
NCHUNK = EPT // K
NB = 5
NBUF = 4
RPT = NP // NS
ZCH = [80] * (RPT // 80)


def _make_seg_sum():
  mesh = plsc.VectorSubcoreMesh(core_axis_name="c", subcore_axis_name="s",
                                num_cores=NC, num_subcores=NS)
  scratch = [
      pltpu.VMEM((2, NB, K), jnp.int32),
      pltpu.VMEM((2, NB, K), jnp.int32),
      pltpu.VMEM((NBUF, K, D), jnp.float32),
      pltpu.VMEM_SHARED((NP, D), jnp.float32),
  ] + [pltpu.SemaphoreType.DMA] * (2 * NBUF + 2)

  def body(x_hbm, ei_hbm, zrow_hbm, part_hbm,
           src_v, dst_v, rows_v, acc, *sems):
    c = lax.axis_index("c")
    s = lax.axis_index("s")
    wid = s * NC + c
    gsems = sems[:NBUF]
    ssems = sems[NBUF:2 * NBUF]
    isems = sems[2 * NBUF:]

    pass

    def iload(blk):
      p = blk % 2
      pltpu.async_copy(ei_hbm.at[0, wid, blk], src_v.at[p], isems[p])
      pltpu.async_copy(ei_hbm.at[1, wid, blk], dst_v.at[p], isems[p])

    def iwait(blk):
      p = blk % 2
      pltpu.make_async_copy(ei_hbm.at[0, wid, 0], src_v.at[p],
                            isems[p]).wait()
      pltpu.make_async_copy(ei_hbm.at[1, wid, 0], dst_v.at[p],
                            isems[p]).wait()

    def gather(i, b):
      blk, loc = divmod(i, NB)
      pltpu.async_copy(x_hbm.at[src_v.at[blk % 2, loc]], rows_v.at[b],
                       gsems[b])

    def gwait(b):
      pltpu.make_async_copy(x_hbm.at[src_v.at[0, 0]], rows_v.at[b],
                            gsems[b]).wait()

    def scatter(i, b):
      blk, loc = divmod(i, NB)
      pltpu.async_copy(rows_v.at[b], acc.at[dst_v.at[blk % 2, loc]],
                       ssems[b], add=True)

    def swait(b):
      pltpu.make_async_copy(rows_v.at[b], acc.at[dst_v.at[0, 0]],
                            ssems[b]).wait()

    iload(0)
    iwait(0)
    iload(1)
    for b in range(NBUF - 1):
      gather(b, b)
    pltpu.sync_copy(zrow_hbm, rows_v.at[NBUF - 1])
    r = 0
    for z in ZCH:
      pltpu.sync_copy(rows_v.at[NBUF - 1, pl.ds(0, z)],
                      acc.at[pl.ds(s * RPT + r, z)])
      r += z
    plsc.subcore_barrier()
    for i in range(NCHUNK):
      b = i % NBUF
      gwait(b)
      scatter(i, b)
      g = i + NBUF - 1
      if g < NCHUNK:
        if g % NB == 0:
          iwait(g // NB)
        if i > 0:
          swait(g % NBUF)
        if i % NB == 0 and 1 < i // NB + 1 < NCHUNK // NB:
          iload(i // NB + 1)
        gather(g, g % NBUF)
    for i in range(NCHUNK - NBUF, NCHUNK):
      swait(i % NBUF)

    plsc.subcore_barrier()

    r = 0
    for j, z in enumerate(ZCH):
      b = j % 2
      if j >= 2:
        pltpu.make_async_copy(rows_v.at[b, pl.ds(0, z)],
                              part_hbm.at[c, pl.ds(s * RPT, z)],
                              gsems[b]).wait()
      pltpu.sync_copy(acc.at[pl.ds(s * RPT + r, z)],
                      rows_v.at[b, pl.ds(0, z)])
      pltpu.async_copy(rows_v.at[b, pl.ds(0, z)],
                       part_hbm.at[c, pl.ds(s * RPT + r, z)], gsems[b])
      r += z
    for j in range(len(ZCH) - 2, len(ZCH)):
      b = j % 2
      pltpu.make_async_copy(rows_v.at[b, pl.ds(0, ZCH[j])],
                            part_hbm.at[c, pl.ds(s * RPT, ZCH[j])],
                            gsems[b]).wait()

  return pl.kernel(
      body,
      out_type=jax.ShapeDtypeStruct((NC, NP, D), jnp.float32),
      mesh=mesh,
      scratch_types=scratch,
  )


@functools.lru_cache(maxsize=None)
def _get_seg_sum():
  return _make_seg_sum()


BE = 4096
EP = 327680
BM = 2000


def _deg_body(dr_ref, o_ref):
  dr = dr_ref[...]
  sub = lax.broadcasted_iota(jnp.int32, (128, BE), 0)
  qrow = lax.shift_right_logical(dr, 7)
  a_t = jnp.where(sub == qrow, 1.0, 0.0).astype(jnp.bfloat16)
  rcol = lax.bitwise_and(dr, 127)
  b_t = jnp.where(sub == rcol, 1.0, 0.0).astype(jnp.bfloat16)
  inc = lax.dot_general(a_t, b_t, (((1,), (1,)), ((), ())),
                        preferred_element_type=jnp.float32)

  @pl.when(pl.program_id(0) == 0)
  def _():
    o_ref[...] = jnp.zeros_like(o_ref)

  o_ref[...] += inc


_deg_tc = pl.pallas_call(
    _deg_body,
    grid=(EP // BE,),
    in_specs=[pl.BlockSpec((1, BE), lambda i: (0, i))],
    out_specs=pl.BlockSpec((128, 128), lambda i: (0, 0)),
    out_shape=jax.ShapeDtypeStruct((128, 128), jnp.float32),
)


def _l1_body(x_ref, p_ref, dg_ref, wl_ref, wr_ref, b_ref, o_ref):
  ssum = p_ref[0] + p_ref[1]
  deg = dg_ref[...]
  agg = ssum * (1.0 / jnp.maximum(deg, 1.0))
  y = (jnp.dot(agg, wl_ref[...], preferred_element_type=jnp.float32,
               precision=lax.Precision.HIGHEST)
       + jnp.dot(x_ref[...], wr_ref[...], preferred_element_type=jnp.float32,
                 precision=lax.Precision.HIGHEST)
       + b_ref[...])
  o_ref[...] = jnp.maximum(y, 0.0)


def _l2_body(h_ref, p_ref, dg_ref, wl_ref, wr_ref, b_ref, wo_ref, bo_ref,
             o_ref):
  ssum = p_ref[0] + p_ref[1]
  deg = dg_ref[...]
  agg = ssum * (1.0 / jnp.maximum(deg, 1.0))
  y = (jnp.dot(agg, wl_ref[...], preferred_element_type=jnp.float32,
               precision=lax.Precision.HIGHEST)
       + jnp.dot(h_ref[...], wr_ref[...], preferred_element_type=jnp.float32,
                 precision=lax.Precision.HIGHEST)
       + b_ref[...])
  h2 = jnp.maximum(y, 0.0)
  o_ref[...] = (jnp.dot(h2, wo_ref[...], preferred_element_type=jnp.float32,
                        precision=lax.Precision.HIGHEST)
                + bo_ref[...])


_row_spec = pl.BlockSpec((BM, D), lambda i: (i, 0))
_part_spec = pl.BlockSpec((NC, BM, D), lambda i: (0, i, 0))
_deg_spec = pl.BlockSpec((BM, 1), lambda i: (i, 0))
_w_spec = pl.BlockSpec((D, D), lambda i: (0, 0))
_b_spec = pl.BlockSpec((1, D), lambda i: (0, 0))

_layer1_tc = pl.pallas_call(
    _l1_body,
    grid=(N_NODES // BM,),
    in_specs=[_row_spec, _part_spec, _deg_spec, _w_spec, _w_spec, _b_spec],
    out_specs=_row_spec,
    out_shape=jax.ShapeDtypeStruct((N_NODES, D), jnp.float32),
)

_layer2_tc = pl.pallas_call(
    _l2_body,
    grid=(N_NODES // BM,),
    in_specs=[_row_spec, _part_spec, _deg_spec, _w_spec, _w_spec, _b_spec,
              _w_spec, _b_spec],
    out_specs=_row_spec,
    out_shape=jax.ShapeDtypeStruct((N_NODES, D), jnp.float32),
)


def kernel(x, edge_index, W1l, W1r, b1, W2l, W2r, b2, Wlin, blin):
  zrow = jnp.zeros((K, D), jnp.float32)
  dst_pad = jnp.pad(edge_index[1], (0, EP - N_EDGES), constant_values=NP)
  b1r = b1.reshape(1, D)
  b2r = b2.reshape(1, D)
  blinr = blin.reshape(1, D)

  ei5 = edge_index.reshape(2, NW, NCHUNK // NB, NB, K)
  part1 = _get_seg_sum()(x, ei5, zrow)
  deg = _deg_tc(dst_pad.reshape(1, EP))
  deg_col = deg.reshape(128 * 128, 1)
  h = _layer1_tc(x, part1, deg_col, W1l, W1r, b1r)
  part2 = _get_seg_sum()(h, ei5, zrow)
  return _layer2_tc(h, part2, deg_col, W2l, W2r, b2r, Wlin, blinr)
